# Initial kernel scaffold; baseline (speedup 1.0000x reference)
import jax
import jax.numpy as jnp
from jax.experimental import pallas as pl


def kernel(x, edge_index, edge_attr, batch, params):
    raise NotImplementedError("write your pallas kernel here")


# speedup vs baseline: 1.2037x; 1.2037x over previous
import functools

import jax
import jax.numpy as jnp
from jax import lax
from jax.experimental import pallas as pl
from jax.experimental.pallas import tpu as pltpu
from jax.experimental.pallas import tpu_sc as plsc

_B = 16
_NSC = 2
_NSUB = 16
_CH = 128


def _mesh():
    return plsc.VectorSubcoreMesh(core_axis_name="c", subcore_axis_name="s")


_SC_PARAMS = pltpu.CompilerParams(use_tc_tiling_on_sc=False)


def _make_gather(e_pad, n, dim, per_w, nch):
    @functools.partial(
        pl.kernel,
        out_type=jax.ShapeDtypeStruct((e_pad, dim), jnp.float32),
        mesh=_mesh(),
        compiler_params=_SC_PARAMS,
        scratch_types=[
            pltpu.VMEM((_CH,), jnp.int32),
            pltpu.VMEM((_CH, dim), jnp.float32),
            pltpu.SemaphoreType.DMA,
        ],
    )
    def gather(src_hbm, x_hbm, out_hbm, idx_v, rows_v, sem):
        wid = lax.axis_index("s") * _NSC + lax.axis_index("c")
        base0 = wid * per_w

        @pl.loop(0, nch)
        def _body(i):
            base = pl.multiple_of(base0 + i * _CH, _CH)
            pltpu.sync_copy(src_hbm.at[pl.ds(base, _CH)], idx_v)
            pltpu.async_copy(x_hbm.at[idx_v], rows_v, sem).wait()
            pltpu.sync_copy(rows_v, out_hbm.at[pl.ds(base, _CH)])

    return gather


def _make_scatter(e_pad, nacc, rpt, dim, per_w, nch, with_deg):
    outs = [jax.ShapeDtypeStruct((_NSC, nacc, dim), jnp.float32)]
    scratch = [
        pltpu.VMEM((_CH,), jnp.int32),
        pltpu.VMEM((_CH, dim), jnp.float32),
        pltpu.VMEM((rpt, dim), jnp.float32),
        pltpu.VMEM_SHARED((nacc, dim), jnp.float32),
    ]
    if with_deg:
        outs.append(jax.ShapeDtypeStruct((_NSC, nacc, 16), jnp.float32))
        scratch += [
            pltpu.VMEM((_CH, 16), jnp.float32),
            pltpu.VMEM((rpt, 16), jnp.float32),
            pltpu.VMEM_SHARED((nacc, 16), jnp.float32),
        ]

    @functools.partial(
        pl.kernel,
        out_type=outs if with_deg else outs[0],
        mesh=_mesh(),
        compiler_params=_SC_PARAMS,
        scratch_types=scratch,
    )
    def scatter(*refs):
        if with_deg:
            (dst_hbm, msg_hbm, z32_hbm, z16_hbm, ones_hbm, agg_hbm, deg_hbm,
             idx_v, rows_v, st32, acc_sh, ones_v, st16, dacc_sh) = refs
        else:
            (dst_hbm, msg_hbm, z32_hbm, agg_hbm,
             idx_v, rows_v, st32, acc_sh) = refs

        cid = lax.axis_index("c")
        sid = lax.axis_index("s")
        wid = sid * _NSC + cid
        r0 = sid * rpt

        pltpu.sync_copy(z32_hbm, st32)
        pltpu.sync_copy(st32, acc_sh.at[pl.ds(r0, rpt)])
        if with_deg:
            pltpu.sync_copy(z16_hbm, st16)
            pltpu.sync_copy(st16, dacc_sh.at[pl.ds(r0, rpt)])
            pltpu.sync_copy(ones_hbm, ones_v)
        plsc.subcore_barrier()

        @pl.loop(0, nch)
        def _body(i):
            base = pl.multiple_of(wid * per_w + i * _CH, _CH)
            pltpu.sync_copy(dst_hbm.at[pl.ds(base, _CH)], idx_v)
            pltpu.sync_copy(msg_hbm.at[pl.ds(base, _CH)], rows_v)
            pltpu.sync_copy(rows_v, acc_sh.at[idx_v], add=True)
            if with_deg:
                pltpu.sync_copy(ones_v, dacc_sh.at[idx_v], add=True)

        plsc.subcore_barrier()
        pltpu.sync_copy(acc_sh.at[pl.ds(r0, rpt)], st32)
        pltpu.sync_copy(st32, agg_hbm.at[cid, pl.ds(r0, rpt)])
        if with_deg:
            pltpu.sync_copy(dacc_sh.at[pl.ds(r0, rpt)], st16)
            pltpu.sync_copy(st16, deg_hbm.at[cid, pl.ds(r0, rpt)])

    return scatter


def _sc_gather(src_p, xv, e_pad, per_w, nch):
    n, dim = xv.shape
    return _make_gather(e_pad, n, dim, per_w, nch)(src_p, xv)


def _sc_scatter(dst_p, msg, nacc, rpt, per_w, nch, with_deg):
    e_pad, dim = msg.shape
    f = _make_scatter(e_pad, nacc, rpt, dim, per_w, nch, with_deg)
    z32 = jnp.zeros((rpt, dim), jnp.float32)
    if with_deg:
        z16 = jnp.zeros((rpt, 16), jnp.float32)
        ones = jnp.ones((_CH, 16), jnp.float32)
        return f(dst_p, msg, z32, z16, ones)
    return f(dst_p, msg, z32)


def _make_proj(n, f_in, dim):
    def body(x_ref, w_ref, b_ref, o_ref):
        o_ref[...] = jnp.maximum(
            jnp.dot(x_ref[...], w_ref[...], preferred_element_type=jnp.float32)
            + b_ref[...], 0.0)

    return pl.pallas_call(
        body, out_shape=jax.ShapeDtypeStruct((n, dim), jnp.float32))


def _make_msg(e_pad, e_in, hid, dim, tile):
    def body(ea_ref, xs_ref, w1_ref, b1_ref, w2_ref, b2r_ref, o_ref):
        h = jnp.maximum(
            jnp.dot(ea_ref[...], w1_ref[...], preferred_element_type=jnp.float32)
            + b1_ref[...], 0.0)
        u = jnp.dot(h, w2_ref[...], preferred_element_type=jnp.float32)
        xs = xs_ref[...]
        acc = jnp.dot(xs, b2r_ref[...], preferred_element_type=jnp.float32)
        for d in range(dim):
            acc = acc + xs[:, d:d + 1] * u[:, d * dim:(d + 1) * dim]
        o_ref[...] = acc

    grid = (e_pad // tile,)
    return pl.pallas_call(
        body,
        grid=grid,
        in_specs=[
            pl.BlockSpec((tile, e_in), lambda i: (i, 0)),
            pl.BlockSpec((tile, dim), lambda i: (i, 0)),
            pl.BlockSpec((e_in, hid), lambda i: (0, 0)),
            pl.BlockSpec((1, hid), lambda i: (0, 0)),
            pl.BlockSpec((hid, dim * dim), lambda i: (0, 0)),
            pl.BlockSpec((dim, dim), lambda i: (0, 0)),
        ],
        out_specs=pl.BlockSpec((tile, dim), lambda i: (i, 0)),
        out_shape=jax.ShapeDtypeStruct((e_pad, dim), jnp.float32),
    )


def _make_node(n, nacc, dim):
    def body(ap_ref, dp_ref, x_ref, root_ref, bias_ref, gnw_ref, gnb_ref,
             gna_ref, bc_ref, br_ref, o_ref):
        f32 = jnp.float32
        agg = ap_ref[pl.ds(0, n), :] + ap_ref[pl.ds(nacc, n), :]
        dg = dp_ref[pl.ds(0, n), :] + dp_ref[pl.ds(nacc, n), :]
        deg = jnp.maximum(dg[:, 0:1], 1.0)
        xin = x_ref[...]
        h2 = jnp.maximum(
            agg / deg
            + jnp.dot(xin, root_ref[...], preferred_element_type=f32)
            + bias_ref[...], 0.0)
        oh = (bc_ref[...] == lax.broadcasted_iota(jnp.int32, (n, _B), 1)
              ).astype(f32)
        oht = (br_ref[0:1, :] == lax.broadcasted_iota(jnp.int32, (_B, n), 0)
               ).astype(f32)
        cnt = jnp.maximum(jnp.sum(oht, axis=1, keepdims=True), 1.0)
        mean = jnp.dot(oht, h2, preferred_element_type=f32) / cnt
        xm = h2 - gna_ref[...] * jnp.dot(oh, mean, preferred_element_type=f32)
        var = jnp.dot(oht, xm * xm, preferred_element_type=f32) / cnt
        varb = jnp.dot(oh, var, preferred_element_type=f32)
        h2n = gnw_ref[...] * xm * lax.rsqrt(varb + 1e-5) + gnb_ref[...]
        o_ref[...] = h2n + xin

    return pl.pallas_call(
        body, out_shape=jax.ShapeDtypeStruct((n, dim), jnp.float32))


def _make_s2s(n, dim, steps):
    def body(x_ref, bc_ref, br_ref, wih_ref, whh_ref, bsum_ref, h1_ref,
             hb1_ref, h2r_ref, hb2_ref, o_ref):
        f32 = jnp.float32
        xx = x_ref[...]
        oh = (bc_ref[...] == lax.broadcasted_iota(jnp.int32, (n, _B), 1)
              ).astype(f32)
        oht = (br_ref[0:1, :] == lax.broadcasted_iota(jnp.int32, (_B, n), 0)
               ).astype(f32)
        q_star = jnp.zeros((_B, 2 * dim), f32)
        hc = jnp.zeros((_B, dim), f32)
        cc = jnp.zeros((_B, dim), f32)
        for _ in range(steps):
            gates = (jnp.dot(q_star, wih_ref[...], preferred_element_type=f32)
                     + jnp.dot(hc, whh_ref[...], preferred_element_type=f32)
                     + bsum_ref[...])
            gi = jax.nn.sigmoid(gates[:, 0:dim])
            gf = jax.nn.sigmoid(gates[:, dim:2 * dim])
            gg = jnp.tanh(gates[:, 2 * dim:3 * dim])
            go = jax.nn.sigmoid(gates[:, 3 * dim:4 * dim])
            cc = gf * cc + gi * gg
            hc = go * jnp.tanh(cc)
            qb = jnp.dot(oh, hc, preferred_element_type=f32)
            e = jnp.sum(xx * qb, axis=1, keepdims=True)
            emax = jnp.max(jnp.where(oh > 0, e, -1e30), axis=0, keepdims=True)
            emaxb = jnp.sum(oh * emax, axis=1, keepdims=True)
            ee = jnp.exp(e - emaxb)
            den = jnp.sum(oh * ee, axis=0, keepdims=True)
            denb = jnp.sum(oh * den, axis=1, keepdims=True)
            aw = ee / (denb + 1e-16)
            r = jnp.dot(oht, aw * xx, preferred_element_type=f32)
            q_star = jnp.concatenate([hc, r], axis=1)
        g = jnp.maximum(
            jnp.dot(q_star, h1_ref[...], preferred_element_type=f32)
            + hb1_ref[...], 0.0)
        o_ref[...] = jnp.sum(g * h2r_ref[...], axis=1, keepdims=True) + hb2_ref[...]

    return pl.pallas_call(
        body, out_shape=jax.ShapeDtypeStruct((_B, 1), jnp.float32))


def kernel(x, edge_index, edge_attr, batch, params):
    f32 = jnp.float32
    n, f_in = x.shape
    e, e_in = edge_attr.shape
    dim = params['W0'].shape[1]
    hid = params['layer0']['W1'].shape[1]

    nw = _NSC * _NSUB
    chunk = nw * _CH
    e_pad = ((e + chunk - 1) // chunk) * chunk
    per_w = e_pad // nw
    nch = per_w // _CH
    rpt = (-(-(n + 1) // _NSUB) + 7) // 8 * 8
    nacc = rpt * _NSUB

    pad = e_pad - e
    src_p = jnp.concatenate([edge_index[0], jnp.zeros((pad,), jnp.int32)])
    dst_p = jnp.concatenate([edge_index[1], jnp.full((pad,), n, jnp.int32)])
    ea_p = jnp.concatenate([edge_attr, jnp.zeros((pad, e_in), f32)])
    bc = batch.reshape(n, 1)
    b8 = jnp.broadcast_to(batch.reshape(1, n), (8, n))

    xcur = _make_proj(n, f_in, dim)(x, params['W0'],
                                    params['b0'].reshape(1, dim))

    msg_call = _make_msg(e_pad, e_in, hid, dim, 1024)
    node_call = _make_node(n, nacc, dim)
    degp = None
    for li in range(2):
        l = params['layer%d' % li]
        xs = _sc_gather(src_p, xcur, e_pad, per_w, nch)
        msg = msg_call(ea_p, xs, l['W1'], l['b1'].reshape(1, hid),
                       l['W2'], l['b2'].reshape(dim, dim))
        if degp is None:
            aggp, degp = _sc_scatter(dst_p, msg, nacc, rpt, per_w, nch, True)
        else:
            aggp = _sc_scatter(dst_p, msg, nacc, rpt, per_w, nch, False)
        xcur = node_call(aggp.reshape(2 * nacc, dim),
                         degp.reshape(2 * nacc, 16),
                         xcur, l['root'], l['bias'].reshape(1, dim),
                         l['gn_weight'].reshape(1, dim),
                         l['gn_bias'].reshape(1, dim),
                         l['gn_alpha'].reshape(1, dim), bc, b8)

    out = _make_s2s(n, dim, 3)(
        xcur, bc, b8, params['Wih'].T, params['Whh'].T,
        (params['bih'] + params['bhh']).reshape(1, 4 * dim),
        params['H1'], params['hb1'].reshape(1, dim),
        params['H2'].reshape(1, dim), params['hb2'].reshape(1, 1))
    return out.reshape(-1)


# speedup vs baseline: 2.4924x; 2.0705x over previous
import functools

import jax
import jax.numpy as jnp
from jax import lax
from jax.experimental import pallas as pl
from jax.experimental.pallas import tpu as pltpu
from jax.experimental.pallas import tpu_sc as plsc

_B = 16
_NSC = 2
_NSUB = 16
_CH = 128
_HI = jax.lax.Precision.HIGHEST


def _dot3(a, b):
    f32 = jnp.float32
    a_hi = a.astype(jnp.bfloat16).astype(f32)
    b_hi = b.astype(jnp.bfloat16).astype(f32)
    return (jnp.dot(a_hi, b_hi, preferred_element_type=f32)
            + jnp.dot(a_hi, b - b_hi, preferred_element_type=f32)
            + jnp.dot(a - a_hi, b_hi, preferred_element_type=f32))


def _seg_dot(onehot, y):
    f32 = jnp.float32
    y_hi = y.astype(jnp.bfloat16).astype(f32)
    return (jnp.dot(onehot, y_hi, preferred_element_type=f32)
            + jnp.dot(onehot, y - y_hi, preferred_element_type=f32))


def _mesh():
    return plsc.VectorSubcoreMesh(core_axis_name="c", subcore_axis_name="s")


_SC_PARAMS = pltpu.CompilerParams(use_tc_tiling_on_sc=False)


def _make_gather(e_pad, n, dim, per_w, nch):
    @functools.partial(
        pl.kernel,
        out_type=jax.ShapeDtypeStruct((e_pad, dim), jnp.float32),
        mesh=_mesh(),
        compiler_params=_SC_PARAMS,
        scratch_types=[
            pltpu.VMEM((_CH,), jnp.int32),
            pltpu.VMEM((_CH, dim), jnp.float32),
            pltpu.SemaphoreType.DMA,
        ],
    )
    def gather(src_hbm, x_hbm, out_hbm, idx_v, rows_v, sem):
        wid = lax.axis_index("s") * _NSC + lax.axis_index("c")
        base0 = wid * per_w

        @pl.loop(0, nch)
        def _body(i):
            base = pl.multiple_of(base0 + i * _CH, _CH)
            pltpu.sync_copy(src_hbm.at[pl.ds(base, _CH)], idx_v)
            pltpu.async_copy(x_hbm.at[idx_v], rows_v, sem).wait()
            pltpu.sync_copy(rows_v, out_hbm.at[pl.ds(base, _CH)])

    return gather


def _make_scatter(e_pad, nacc, rpt, dim, per_w, nch, with_deg):
    outs = [jax.ShapeDtypeStruct((_NSC, nacc, dim), jnp.float32)]
    scratch = [
        pltpu.VMEM((_CH,), jnp.int32),
        pltpu.VMEM((_CH, dim), jnp.float32),
        pltpu.VMEM((rpt, dim), jnp.float32),
        pltpu.VMEM_SHARED((nacc, dim), jnp.float32),
    ]
    if with_deg:
        outs.append(jax.ShapeDtypeStruct((_NSC, nacc, 16), jnp.float32))
        scratch += [
            pltpu.VMEM((_CH, 16), jnp.float32),
            pltpu.VMEM((rpt, 16), jnp.float32),
            pltpu.VMEM_SHARED((nacc, 16), jnp.float32),
        ]

    @functools.partial(
        pl.kernel,
        out_type=outs if with_deg else outs[0],
        mesh=_mesh(),
        compiler_params=_SC_PARAMS,
        scratch_types=scratch,
    )
    def scatter(*refs):
        if with_deg:
            (dst_hbm, msg_hbm, z32_hbm, z16_hbm, ones_hbm, agg_hbm, deg_hbm,
             idx_v, rows_v, st32, acc_sh, ones_v, st16, dacc_sh) = refs
        else:
            (dst_hbm, msg_hbm, z32_hbm, agg_hbm,
             idx_v, rows_v, st32, acc_sh) = refs

        cid = lax.axis_index("c")
        sid = lax.axis_index("s")
        wid = sid * _NSC + cid
        r0 = sid * rpt

        pltpu.sync_copy(z32_hbm, st32)
        pltpu.sync_copy(st32, acc_sh.at[pl.ds(r0, rpt)])
        if with_deg:
            pltpu.sync_copy(z16_hbm, st16)
            pltpu.sync_copy(st16, dacc_sh.at[pl.ds(r0, rpt)])
            pltpu.sync_copy(ones_hbm, ones_v)
        plsc.subcore_barrier()

        @pl.loop(0, nch)
        def _body(i):
            base = pl.multiple_of(wid * per_w + i * _CH, _CH)
            pltpu.sync_copy(dst_hbm.at[pl.ds(base, _CH)], idx_v)
            pltpu.sync_copy(msg_hbm.at[pl.ds(base, _CH)], rows_v)
            pltpu.sync_copy(rows_v, acc_sh.at[idx_v], add=True)
            if with_deg:
                pltpu.sync_copy(ones_v, dacc_sh.at[idx_v], add=True)

        plsc.subcore_barrier()
        pltpu.sync_copy(acc_sh.at[pl.ds(r0, rpt)], st32)
        pltpu.sync_copy(st32, agg_hbm.at[cid, pl.ds(r0, rpt)])
        if with_deg:
            pltpu.sync_copy(dacc_sh.at[pl.ds(r0, rpt)], st16)
            pltpu.sync_copy(st16, deg_hbm.at[cid, pl.ds(r0, rpt)])

    return scatter


def _sc_gather(src_p, xv, e_pad, per_w, nch):
    n, dim = xv.shape
    return _make_gather(e_pad, n, dim, per_w, nch)(src_p, xv)


def _sc_scatter(dst_p, msg, nacc, rpt, per_w, nch, with_deg):
    e_pad, dim = msg.shape
    f = _make_scatter(e_pad, nacc, rpt, dim, per_w, nch, with_deg)
    z32 = jnp.zeros((rpt, dim), jnp.float32)
    if with_deg:
        z16 = jnp.zeros((rpt, 16), jnp.float32)
        ones = jnp.ones((_CH, 16), jnp.float32)
        return f(dst_p, msg, z32, z16, ones)
    return f(dst_p, msg, z32)


def _make_proj(n, f_in, dim):
    def body(x_ref, w_ref, b_ref, o_ref):
        x = x_ref[...]
        w = w_ref[...]
        x_hi = x.astype(jnp.bfloat16).astype(jnp.float32)
        w_hi = w.astype(jnp.bfloat16).astype(jnp.float32)
        acc = (jnp.dot(x_hi, w_hi, preferred_element_type=jnp.float32)
               + jnp.dot(x_hi, w - w_hi, preferred_element_type=jnp.float32)
               + jnp.dot(x - x_hi, w_hi, preferred_element_type=jnp.float32))
        o_ref[...] = jnp.maximum(acc + b_ref[...], 0.0)

    return pl.pallas_call(
        body, out_shape=jax.ShapeDtypeStruct((n, dim), jnp.float32))


def _make_msg(e_pad, e_in, hid, dim, tile):
    def body(eat_ref, xs_ref, w1t_ref, b1c_ref, w2h_ref, w2l_ref,
             b2rt_ref, o_ref):
        f32 = jnp.float32
        ht = jnp.maximum(_dot3(w1t_ref[...], eat_ref[...]) + b1c_ref[...],
                         0.0)
        ht_hi = ht.astype(jnp.bfloat16).astype(f32)
        ht_lo = ht - ht_hi
        w2h = w2h_ref[...]
        ut = (jnp.dot(w2h, ht_hi, preferred_element_type=f32)
              + jnp.dot(w2h, ht_lo, preferred_element_type=f32)
              + jnp.dot(w2l_ref[...], ht_hi, preferred_element_type=f32))
        xst = xs_ref[...].T
        acct = _dot3(b2rt_ref[...], xst)
        for d in range(dim):
            acct = acct + ut[d * dim:(d + 1) * dim, :] * xst[d:d + 1, :]
        o_ref[...] = acct.T

    grid = (e_pad // tile,)
    return pl.pallas_call(
        body,
        grid=grid,
        in_specs=[
            pl.BlockSpec((e_in, tile), lambda i: (0, i)),
            pl.BlockSpec((tile, dim), lambda i: (i, 0)),
            pl.BlockSpec((hid, e_in), lambda i: (0, 0)),
            pl.BlockSpec((hid, 1), lambda i: (0, 0)),
            pl.BlockSpec((dim * dim, hid), lambda i: (0, 0)),
            pl.BlockSpec((dim * dim, hid), lambda i: (0, 0)),
            pl.BlockSpec((dim, dim), lambda i: (0, 0)),
        ],
        out_specs=pl.BlockSpec((tile, dim), lambda i: (i, 0)),
        out_shape=jax.ShapeDtypeStruct((e_pad, dim), jnp.float32),
    )


def _make_node_a(n, nacc, dim):
    def body(ap_ref, dp_ref, x_ref, root_ref, bias_ref, o_ref):
        f32 = jnp.float32
        agg = ap_ref[pl.ds(0, n), :] + ap_ref[pl.ds(nacc, n), :]
        dg = dp_ref[pl.ds(0, n), :] + dp_ref[pl.ds(nacc, n), :]
        deg = jnp.maximum(dg[:, 0:1], 1.0)
        o_ref[...] = jnp.maximum(
            agg / deg
            + _dot3(x_ref[...], root_ref[...])
            + bias_ref[...], 0.0)

    return pl.pallas_call(
        body, out_shape=jax.ShapeDtypeStruct((n, dim), jnp.float32))


def _make_node_b(n, dim):
    def body(h2_ref, x_ref, gnw_ref, gnb_ref, gna_ref, bc_ref, br_ref,
             o_ref):
        f32 = jnp.float32
        h2 = h2_ref[...]
        oh = (bc_ref[...] == lax.broadcasted_iota(jnp.int32, (n, _B), 1)
              ).astype(f32)
        oht = (br_ref[0:1, :] == lax.broadcasted_iota(jnp.int32, (_B, n), 0)
               ).astype(f32)
        cnt = jnp.maximum(jnp.sum(oht, axis=1, keepdims=True), 1.0)
        mean = _seg_dot(oht, h2) / cnt
        xm = h2 - gna_ref[...] * _seg_dot(oh, mean)
        var = _seg_dot(oht, xm * xm) / cnt
        varb = _seg_dot(oh, var)
        h2n = gnw_ref[...] * xm / jnp.sqrt(varb + 1e-5) + gnb_ref[...]
        o_ref[...] = h2n + x_ref[...]

    return pl.pallas_call(
        body, out_shape=jax.ShapeDtypeStruct((n, dim), jnp.float32))


def _make_s2s(n, dim, steps):
    def body(x_ref, bc_ref, br_ref, wih_ref, whh_ref, bsum_ref, h1_ref,
             hb1_ref, h2r_ref, hb2_ref, o_ref):
        f32 = jnp.float32
        xx = x_ref[...]
        oh = (bc_ref[...] == lax.broadcasted_iota(jnp.int32, (n, _B), 1)
              ).astype(f32)
        oht = (br_ref[0:1, :] == lax.broadcasted_iota(jnp.int32, (_B, n), 0)
               ).astype(f32)
        q_star = jnp.zeros((_B, 2 * dim), f32)
        hc = jnp.zeros((_B, dim), f32)
        cc = jnp.zeros((_B, dim), f32)
        for _ in range(steps):
            gates = (_dot3(q_star, wih_ref[...]) + _dot3(hc, whh_ref[...])
                     + bsum_ref[...])
            gi = jax.nn.sigmoid(gates[:, 0:dim])
            gf = jax.nn.sigmoid(gates[:, dim:2 * dim])
            gg = jnp.tanh(gates[:, 2 * dim:3 * dim])
            go = jax.nn.sigmoid(gates[:, 3 * dim:4 * dim])
            cc = gf * cc + gi * gg
            hc = go * jnp.tanh(cc)
            qb = _seg_dot(oh, hc)
            e = jnp.sum(xx * qb, axis=1, keepdims=True)
            emax = jnp.max(jnp.where(oh > 0, e, -1e30), axis=0, keepdims=True)
            emaxb = jnp.sum(oh * emax, axis=1, keepdims=True)
            ee = jnp.exp(e - emaxb)
            den = jnp.sum(oh * ee, axis=0, keepdims=True)
            denb = jnp.sum(oh * den, axis=1, keepdims=True)
            aw = ee / (denb + 1e-16)
            r = _seg_dot(oht, aw * xx)
            q_star = jnp.concatenate([hc, r], axis=1)
        g = jnp.maximum(_dot3(q_star, h1_ref[...]) + hb1_ref[...], 0.0)
        o_ref[...] = jnp.sum(g * h2r_ref[...], axis=1, keepdims=True) + hb2_ref[...]

    return pl.pallas_call(
        body, out_shape=jax.ShapeDtypeStruct((_B, 1), jnp.float32))


def kernel(x, edge_index, edge_attr, batch, params):
    f32 = jnp.float32
    n, f_in = x.shape
    e, e_in = edge_attr.shape
    dim = params['W0'].shape[1]
    hid = params['layer0']['W1'].shape[1]

    nw = _NSC * _NSUB
    chunk = nw * _CH
    e_pad = ((e + chunk - 1) // chunk) * chunk
    per_w = e_pad // nw
    nch = per_w // _CH
    rpt = (-(-(n + 1) // _NSUB) + 7) // 8 * 8
    nacc = rpt * _NSUB

    pad = e_pad - e
    src_p = jnp.concatenate([edge_index[0], jnp.zeros((pad,), jnp.int32)])
    dst_p = jnp.concatenate([edge_index[1], jnp.full((pad,), n, jnp.int32)])
    eat_p = jnp.concatenate([edge_attr, jnp.zeros((pad, e_in), f32)]).T
    bc = batch.reshape(n, 1)
    b8 = jnp.broadcast_to(batch.reshape(1, n), (8, n))

    xcur = _make_proj(n, f_in, dim)(x, params['W0'],
                                    params['b0'].reshape(1, dim))

    msg_call = _make_msg(e_pad, e_in, hid, dim, 1024)
    node_a = _make_node_a(n, nacc, dim)
    node_b = _make_node_b(n, dim)
    degp = None
    for li in range(2):
        l = params['layer%d' % li]
        xs = _sc_gather(src_p, xcur, e_pad, per_w, nch)
        w2t = l['W2'].T
        w2t_hi = w2t.astype(jnp.bfloat16).astype(f32)
        msg = msg_call(eat_p, xs, l['W1'].T, l['b1'].reshape(hid, 1),
                       w2t_hi, w2t - w2t_hi, l['b2'].reshape(dim, dim).T)
        if degp is None:
            aggp, degp = _sc_scatter(dst_p, msg, nacc, rpt, per_w, nch, True)
        else:
            aggp = _sc_scatter(dst_p, msg, nacc, rpt, per_w, nch, False)
        h2 = node_a(aggp.reshape(2 * nacc, dim), degp.reshape(2 * nacc, 16),
                    xcur, l['root'], l['bias'].reshape(1, dim))
        xcur = node_b(h2, xcur, l['gn_weight'].reshape(1, dim),
                      l['gn_bias'].reshape(1, dim),
                      l['gn_alpha'].reshape(1, dim), bc, b8)

    out = _make_s2s(n, dim, 3)(
        xcur, bc, b8, params['Wih'].T, params['Whh'].T,
        (params['bih'] + params['bhh']).reshape(1, 4 * dim),
        params['H1'], params['hb1'].reshape(1, dim),
        params['H2'].reshape(1, dim), params['hb2'].reshape(1, 1))
    return out.reshape(-1)
